# baseline (speedup 1.0000x reference)
import functools

import jax
import jax.numpy as jnp
from jax import lax
from jax.experimental import pallas as pl
from jax.experimental.pallas import tpu as pltpu
from jax.experimental.pallas import tpu_sc as plsc

_NW = 32
_CHUNK = 128


@functools.partial(jax.jit, static_argnames=())
def kernel(c, table):
    B, = c.shape
    K, D2 = table.shape
    D = D2 // 2
    n_chunks = B // (_NW * _CHUNK)
    G = _NW * n_chunks
    c2 = c.reshape(G, _CHUNK).astype(jnp.int32)

    mesh = plsc.VectorSubcoreMesh(core_axis_name="core", subcore_axis_name="sub")

    @functools.partial(
        pl.kernel,
        mesh=mesh,
        compiler_params=pltpu.CompilerParams(use_tc_tiling_on_sc=False),
        out_type=(
            jax.ShapeDtypeStruct((G, _CHUNK, D), jnp.float32),
            jax.ShapeDtypeStruct((G, _CHUNK, D), jnp.float32),
        ),
        scratch_types=[
            pltpu.VMEM((n_chunks, _CHUNK), jnp.int32),
            pltpu.VMEM((n_chunks, _CHUNK, D2), jnp.float32),
            pltpu.SemaphoreType.DMA,
        ],
    )
    def _gather_split(idx_hbm, table_hbm, mu_hbm, lv_hbm, idx_v, rows_v, sem):
        wid = lax.axis_index("sub") * 2 + lax.axis_index("core")
        base = wid * n_chunks
        pltpu.sync_copy(idx_hbm.at[pl.ds(base, n_chunks)], idx_v)
        copies = [
            pltpu.async_copy(table_hbm.at[idx_v.at[j]], rows_v.at[j], sem)
            for j in range(n_chunks)
        ]
        for j in range(n_chunks):
            copies[j].wait()
            pltpu.sync_copy(rows_v.at[j, :, pl.ds(0, D)], mu_hbm.at[base + j])
            pltpu.sync_copy(rows_v.at[j, :, pl.ds(D, D)], lv_hbm.at[base + j])

    mu, lv = _gather_split(c2, table)
    return mu.reshape(B, D), lv.reshape(B, D)


# speedup vs baseline: 1.5258x; 1.5258x over previous
import functools

import jax
import jax.numpy as jnp
from jax import lax
from jax.experimental import pallas as pl
from jax.experimental.pallas import tpu as pltpu
from jax.experimental.pallas import tpu_sc as plsc

_NW = 32
_NB = 16


def kernel(c, table):
    B, = c.shape
    K, D2 = table.shape
    D = D2 // 2
    bpw = B // _NW

    c2 = c.reshape(_NW, bpw).astype(jnp.int32)
    mesh = plsc.VectorSubcoreMesh(core_axis_name="core", subcore_axis_name="sub")

    @functools.partial(
        pl.kernel,
        mesh=mesh,
        compiler_params=pltpu.CompilerParams(needs_layout_passes=False),
        out_type=(
            jax.ShapeDtypeStruct((D, B), jnp.float32),
            jax.ShapeDtypeStruct((D, B), jnp.float32),
        ),
        scratch_types=[
            pltpu.VMEM((1, bpw), jnp.int32),
            pltpu.VMEM((_NB, 8, D2), jnp.float32),
            pltpu.VMEM((D2, bpw), jnp.float32),
            pltpu.SemaphoreType.DMA,
        ],
    )
    def _gather_split(idx_hbm, tv_hbm, mu_hbm, lv_hbm, idx_v, blk_v, stage_v, sem):
        wid = lax.axis_index("sub") * 2 + lax.axis_index("core")
        pltpu.sync_copy(idx_hbm.at[pl.ds(wid, 1)], idx_v)
        zeros = jnp.zeros((16,), jnp.int32)
        ar16 = jnp.arange(16, dtype=jnp.int32)

        def batch(bi, _):
            vec = idx_v[0, pl.ds(bi * _NB, _NB)]
            blks = vec >> 3
            for lane in range(_NB):
                pltpu.async_copy(
                    tv_hbm.at[pl.ds(blks[lane] * 8, 8)], blk_v.at[lane], sem)
            for lane in range(_NB):
                pltpu.make_async_copy(
                    tv_hbm.at[pl.ds(0, 8)], blk_v.at[lane], sem).wait()
            for lane in range(_NB):
                e = vec[lane] & 7
                ii = bi * _NB + lane
                for g in range(D2 // 16):
                    vals = blk_v[lane, e, pl.ds(g * 16, 16)]
                    d = g * 16 + ar16
                    plsc.store_scatter(stage_v, [d, zeros + ii], vals)
            return ()

        lax.fori_loop(0, bpw // _NB, batch, ())
        pltpu.sync_copy(stage_v.at[pl.ds(0, D)],
                        mu_hbm.at[:, pl.ds(wid * bpw, bpw)])
        pltpu.sync_copy(stage_v.at[pl.ds(D, D)],
                        lv_hbm.at[:, pl.ds(wid * bpw, bpw)])

    mu, lv = _gather_split(c2, table)
    return mu.T, lv.T
